# baseline (speedup 1.0000x reference)
import functools

import jax
import jax.numpy as jnp
from jax import lax
from jax.experimental import pallas as pl
from jax.experimental.pallas import tpu as pltpu
from jax.experimental.pallas import tpu_sc as plsc

NC = 2
NS = 16
NW = NC * NS
L = 16
EB = 80
EPAD = 327680
RW = EPAD // (NW * EB)
CH = 32
RW0 = RW
RW1 = 2 * RW - RW0
GPAD = 16
JPAD = 112


def _mesh():
  return plsc.VectorSubcoreMesh(core_axis_name="c", subcore_axis_name="s")


def _deg_build(N):
  NV = EPAD // NW // L
  DCH = 2048
  NCHUNK = EPAD // NW // DCH
  ND = N + JPAD

  @functools.partial(
      pl.kernel,
      out_type=jax.ShapeDtypeStruct((NW, ND), jnp.float32),
      mesh=_mesh(),
      compiler_params=pltpu.CompilerParams(needs_layout_passes=False),
      scratch_types=[
          pltpu.VMEM((DCH,), jnp.int32),
          pltpu.VMEM((ND,), jnp.float32),
      ],
  )
  def deg_kernel(dst_hbm, out_hbm, idx_v, deg_v):
    cid = lax.axis_index("c")
    sid = lax.axis_index("s")
    wid = sid * NC + cid

    zv = jnp.zeros((L,), jnp.float32)

    def zbody(i, carry):
      deg_v[pl.ds(i * L, L)] = zv
      return carry

    lax.fori_loop(0, ND // L, zbody, 0)

    ones = jnp.ones((L,), jnp.float32)

    def cbody(c, carry):
      pltpu.sync_copy(dst_hbm.at[wid, pl.ds(c * DCH, DCH)], idx_v)

      def ebody(j, carry2):
        idx = idx_v[pl.ds(j * L, L)]
        plsc.addupdate_scatter(deg_v, [idx], ones)
        return carry2

      lax.fori_loop(0, DCH // L, ebody, 0)
      return carry

    lax.fori_loop(0, NCHUNK, cbody, 0)
    pltpu.sync_copy(deg_v, out_hbm.at[wid])

  return deg_kernel


def _msg_build(N, D):
  RS = 8 * ((N + JPAD + 8 * NS - 1) // (8 * NS))
  NP = RS * NS
  ZR = 8

  @functools.partial(
      pl.kernel,
      out_type=jax.ShapeDtypeStruct((NC, NP, D), jnp.float32),
      mesh=_mesh(),
      compiler_params=pltpu.CompilerParams(needs_layout_passes=False),
      scratch_types=[
          pltpu.VMEM((CH, EB), jnp.int32),
          pltpu.VMEM((CH, EB), jnp.int32),
          pltpu.VMEM((2, EB, D), jnp.float32),
          pltpu.VMEM((ZR, D), jnp.float32),
          pltpu.VMEM_SHARED((NP, D), jnp.float32),
          [pltpu.SemaphoreType.DMA] * 2,
          [pltpu.SemaphoreType.DMA] * 2,
      ],
  )
  def msg_kernel(g_hbm, src_hbm, dst_hbm, out_hbm,
                 src_v, dst_v, gbuf, zbuf, acc, gsem, ssem):
    cid = lax.axis_index("c")
    sid = lax.axis_index("s")
    rw = RW1 + (1 - cid) * (RW0 - RW1)
    base = cid * (NS * RW0) + sid * rw

    zv = jnp.zeros((L,), jnp.float32)

    def zbody(i, carry):
      for kk in range(D // L):
        zbuf[i, pl.ds(kk * L, L)] = zv
      return carry

    lax.fori_loop(0, ZR, zbody, 0)

    def zcopy(t, carry):
      pltpu.sync_copy(zbuf, acc.at[pl.ds(sid * RS + t * ZR, ZR)])
      return carry

    lax.fori_loop(0, RS // ZR, zcopy, 0)
    plsc.subcore_barrier()

    NB = 2

    def cbody(c, carry):
      pltpu.sync_copy(src_hbm.at[pl.ds(base + c * CH, CH)], src_v)
      pltpu.sync_copy(dst_hbm.at[pl.ds(base + c * CH, CH)], dst_v)
      for b in range(NB):
        pltpu.async_copy(g_hbm.at[src_v.at[b]], gbuf.at[b], gsem[b])

      def qbody(q, carry2):
        j0 = NB * q
        for b in range(NB):
          pltpu.make_async_copy(g_hbm.at[src_v.at[j0 + b]], gbuf.at[b],
                                gsem[b]).wait()
          pltpu.async_copy(gbuf.at[b], acc.at[dst_v.at[j0 + b]], ssem[b],
                           add=True)
        for b in range(NB):
          jn = j0 + NB + b

          @pl.when(jn < CH)
          def _():
            pltpu.make_async_copy(gbuf.at[b], acc.at[dst_v.at[j0 + b]],
                                  ssem[b]).wait()
            pltpu.async_copy(g_hbm.at[src_v.at[jn]], gbuf.at[b], gsem[b])

        return carry2

      lax.fori_loop(0, CH // NB, qbody, 0)
      for b in range(NB):
        pltpu.make_async_copy(gbuf.at[b], acc.at[dst_v.at[CH - NB + b]],
                              ssem[b]).wait()
      return carry

    lax.fori_loop(0, rw // CH, cbody, 0)

    plsc.subcore_barrier()
    pltpu.sync_copy(acc.at[pl.ds(sid * RS, RS)],
                    out_hbm.at[cid, pl.ds(sid * RS, RS)])

  return msg_kernel


def _tc1(degp_ref, x_ref, w1_ref, dinv_ref, g1_ref):
  n = x_ref.shape[0]
  deg = 1.0 + jnp.sum(degp_ref[...], axis=0)[:n]
  dinv = lax.rsqrt(deg)[:, None]
  dinv_ref[...] = dinv
  g = jnp.dot(x_ref[...], w1_ref[...],
              preferred_element_type=jnp.float32) * dinv
  g1_ref[...] = jnp.concatenate(
      [g, jnp.zeros((GPAD, g.shape[1]), jnp.float32)], axis=0)


def _tc2(s_ref, g_ref, dinv_ref, b_ref, w_ref, gn_ref):
  n = dinv_ref.shape[0]
  dinv = dinv_ref[...]
  h = jnp.maximum(
      dinv * (s_ref[0, :n] + s_ref[1, :n] + g_ref[:n]) + b_ref[...], 0.0)
  g = jnp.dot(h, w_ref[...], preferred_element_type=jnp.float32) * dinv
  gn_ref[...] = jnp.concatenate(
      [g, jnp.zeros((GPAD, g.shape[1]), jnp.float32)], axis=0)


def _tc3(s_ref, g_ref, dinv_ref, b_ref, wf1_ref, bf1_ref, wf2_ref, bf2_ref,
         out_ref):
  n = dinv_ref.shape[0]
  dinv = dinv_ref[...]
  h2 = jnp.maximum(
      dinv * (s_ref[0, :n] + s_ref[1, :n] + g_ref[:n]) + b_ref[...], 0.0)
  h3 = jnp.maximum(
      jnp.dot(h2, wf1_ref[...], preferred_element_type=jnp.float32)
      + bf1_ref[...], 0.0)
  o = jnp.dot(h3, wf2_ref[...], preferred_element_type=jnp.float32) + bf2_ref[...]
  nrm = jnp.sqrt(jnp.sum(o * o))
  out_ref[...] = o / jnp.maximum(nrm, 1e-12)


def kernel(x, edge_index, W1, b1, W2, b2, Wf1, bf1, Wf2, bf2):
  N, D = x.shape
  E = edge_index.shape[1]
  F = Wf1.shape[1]

  ar = jnp.arange(EPAD - E, dtype=jnp.int32)
  pad_src = (ar * 7) % N
  pad_dst = N + (ar % JPAD)
  src2 = jnp.concatenate([edge_index[0], pad_src]).reshape(NW, RW * EB)
  dst2 = jnp.concatenate([edge_index[1], pad_dst]).reshape(NW, RW * EB)
  src3 = src2.reshape(NW * RW, EB)
  dst3 = dst2.reshape(NW * RW, EB)

  degp = _deg_build(N)(dst2)

  dinv, g1 = pl.pallas_call(
      _tc1,
      out_shape=(jax.ShapeDtypeStruct((N, 1), jnp.float32),
                 jax.ShapeDtypeStruct((N + GPAD, D), jnp.float32)),
  )(degp, x, W1)

  msg = _msg_build(N, D)
  s1 = msg(g1, src3, dst3)

  g2 = pl.pallas_call(
      _tc2,
      out_shape=jax.ShapeDtypeStruct((N + GPAD, D), jnp.float32),
  )(s1, g1, dinv, b1.reshape(1, D), W2)

  s2 = msg(g2, src3, dst3)

  out = pl.pallas_call(
      _tc3,
      out_shape=jax.ShapeDtypeStruct((N, 1), jnp.float32),
  )(s2, g2, dinv, b2.reshape(1, D), Wf1, bf1.reshape(1, F), Wf2,
    bf2.reshape(1, 1))
  return out


# speedup vs baseline: 1.0748x; 1.0748x over previous
import functools

import jax
import jax.numpy as jnp
from jax import lax
from jax.experimental import pallas as pl
from jax.experimental.pallas import tpu as pltpu
from jax.experimental.pallas import tpu_sc as plsc

NC = 2
NS = 16
NW = NC * NS
L = 16
EB = 16
EPAD = 327680
RW = EPAD // (NW * EB)
CH = 64
RW0 = RW
RW1 = 2 * RW - RW0
GPAD = 16
JPAD = 112


def _mesh():
  return plsc.VectorSubcoreMesh(core_axis_name="c", subcore_axis_name="s")


def _deg_build(N):
  NV = EPAD // NW // L
  DCH = 2048
  NCHUNK = EPAD // NW // DCH
  ND = N + JPAD

  @functools.partial(
      pl.kernel,
      out_type=jax.ShapeDtypeStruct((NW, ND), jnp.float32),
      mesh=_mesh(),
      compiler_params=pltpu.CompilerParams(needs_layout_passes=False),
      scratch_types=[
          pltpu.VMEM((DCH,), jnp.int32),
          pltpu.VMEM((ND,), jnp.float32),
      ],
  )
  def deg_kernel(dst_hbm, out_hbm, idx_v, deg_v):
    cid = lax.axis_index("c")
    sid = lax.axis_index("s")
    wid = sid * NC + cid

    zv = jnp.zeros((L,), jnp.float32)

    def zbody(i, carry):
      deg_v[pl.ds(i * L, L)] = zv
      return carry

    lax.fori_loop(0, ND // L, zbody, 0)

    ones = jnp.ones((L,), jnp.float32)

    def cbody(c, carry):
      pltpu.sync_copy(dst_hbm.at[wid, pl.ds(c * DCH, DCH)], idx_v)

      def ebody(j, carry2):
        idx = idx_v[pl.ds(j * L, L)]
        plsc.addupdate_scatter(deg_v, [idx], ones)
        return carry2

      lax.fori_loop(0, DCH // L, ebody, 0)
      return carry

    lax.fori_loop(0, NCHUNK, cbody, 0)
    pltpu.sync_copy(deg_v, out_hbm.at[wid])

  return deg_kernel


def _msg_build(N, D):
  RS = 8 * ((N + JPAD + 8 * NS - 1) // (8 * NS))
  NP = RS * NS
  ZR = 8

  @functools.partial(
      pl.kernel,
      out_type=jax.ShapeDtypeStruct((NC, NP, D), jnp.float32),
      mesh=_mesh(),
      compiler_params=pltpu.CompilerParams(needs_layout_passes=False),
      scratch_types=[
          pltpu.VMEM((CH, EB), jnp.int32),
          pltpu.VMEM((CH, EB), jnp.int32),
          pltpu.VMEM((8, EB, D), jnp.float32),
          pltpu.VMEM((ZR, D), jnp.float32),
          pltpu.VMEM_SHARED((NP, D), jnp.float32),
          [pltpu.SemaphoreType.DMA] * 8,
          [pltpu.SemaphoreType.DMA] * 8,
      ],
  )
  def msg_kernel(g_hbm, src_hbm, dst_hbm, out_hbm,
                 src_v, dst_v, gbuf, zbuf, acc, gsem, ssem):
    cid = lax.axis_index("c")
    sid = lax.axis_index("s")
    rw = RW1 + (1 - cid) * (RW0 - RW1)
    base = cid * (NS * RW0) + sid * rw

    zv = jnp.zeros((L,), jnp.float32)

    def zbody(i, carry):
      for kk in range(D // L):
        zbuf[i, pl.ds(kk * L, L)] = zv
      return carry

    lax.fori_loop(0, ZR, zbody, 0)

    def zcopy(t, carry):
      pltpu.sync_copy(zbuf, acc.at[pl.ds(sid * RS + t * ZR, ZR)])
      return carry

    lax.fori_loop(0, RS // ZR, zcopy, 0)
    plsc.subcore_barrier()

    NB = 8

    def cbody(c, carry):
      pltpu.sync_copy(src_hbm.at[pl.ds(base + c * CH, CH)], src_v)
      pltpu.sync_copy(dst_hbm.at[pl.ds(base + c * CH, CH)], dst_v)
      for b in range(NB):
        pltpu.async_copy(g_hbm.at[src_v.at[b]], gbuf.at[b], gsem[b])

      def qbody(q, carry2):
        j0 = NB * q
        for b in range(NB):
          pltpu.make_async_copy(g_hbm.at[src_v.at[j0 + b]], gbuf.at[b],
                                gsem[b]).wait()
          pltpu.async_copy(gbuf.at[b], acc.at[dst_v.at[j0 + b]], ssem[b],
                           add=True)
        for b in range(NB):
          jn = j0 + NB + b

          @pl.when(jn < CH)
          def _():
            pltpu.make_async_copy(gbuf.at[b], acc.at[dst_v.at[j0 + b]],
                                  ssem[b]).wait()
            pltpu.async_copy(g_hbm.at[src_v.at[jn]], gbuf.at[b], gsem[b])

        return carry2

      lax.fori_loop(0, CH // NB, qbody, 0)
      for b in range(NB):
        pltpu.make_async_copy(gbuf.at[b], acc.at[dst_v.at[CH - NB + b]],
                              ssem[b]).wait()
      return carry

    lax.fori_loop(0, rw // CH, cbody, 0)

    plsc.subcore_barrier()
    pltpu.sync_copy(acc.at[pl.ds(sid * RS, RS)],
                    out_hbm.at[cid, pl.ds(sid * RS, RS)])

  return msg_kernel


def _tc1(degp_ref, x_ref, w1_ref, dinv_ref, g1_ref):
  n = x_ref.shape[0]
  deg = 1.0 + jnp.sum(degp_ref[...], axis=0)[:n]
  dinv = lax.rsqrt(deg)[:, None]
  dinv_ref[...] = dinv
  g = jnp.dot(x_ref[...], w1_ref[...],
              preferred_element_type=jnp.float32) * dinv
  g1_ref[...] = jnp.concatenate(
      [g, jnp.zeros((GPAD, g.shape[1]), jnp.float32)], axis=0)


def _tc2(s_ref, g_ref, dinv_ref, b_ref, w_ref, gn_ref):
  n = dinv_ref.shape[0]
  dinv = dinv_ref[...]
  h = jnp.maximum(
      dinv * (s_ref[0, :n] + s_ref[1, :n] + g_ref[:n]) + b_ref[...], 0.0)
  g = jnp.dot(h, w_ref[...], preferred_element_type=jnp.float32) * dinv
  gn_ref[...] = jnp.concatenate(
      [g, jnp.zeros((GPAD, g.shape[1]), jnp.float32)], axis=0)


def _tc3(s_ref, g_ref, dinv_ref, b_ref, wf1_ref, bf1_ref, wf2_ref, bf2_ref,
         out_ref):
  n = dinv_ref.shape[0]
  dinv = dinv_ref[...]
  h2 = jnp.maximum(
      dinv * (s_ref[0, :n] + s_ref[1, :n] + g_ref[:n]) + b_ref[...], 0.0)
  h3 = jnp.maximum(
      jnp.dot(h2, wf1_ref[...], preferred_element_type=jnp.float32)
      + bf1_ref[...], 0.0)
  o = jnp.dot(h3, wf2_ref[...], preferred_element_type=jnp.float32) + bf2_ref[...]
  nrm = jnp.sqrt(jnp.sum(o * o))
  out_ref[...] = o / jnp.maximum(nrm, 1e-12)


def kernel(x, edge_index, W1, b1, W2, b2, Wf1, bf1, Wf2, bf2):
  N, D = x.shape
  E = edge_index.shape[1]
  F = Wf1.shape[1]

  ar = jnp.arange(EPAD - E, dtype=jnp.int32)
  pad_src = (ar * 7) % N
  pad_dst = N + (ar % JPAD)
  src2 = jnp.concatenate([edge_index[0], pad_src]).reshape(NW, RW * EB)
  dst2 = jnp.concatenate([edge_index[1], pad_dst]).reshape(NW, RW * EB)
  src3 = src2.reshape(NW * RW, EB)
  dst3 = dst2.reshape(NW * RW, EB)

  degp = _deg_build(N)(dst2)

  dinv, g1 = pl.pallas_call(
      _tc1,
      out_shape=(jax.ShapeDtypeStruct((N, 1), jnp.float32),
                 jax.ShapeDtypeStruct((N + GPAD, D), jnp.float32)),
  )(degp, x, W1)

  msg = _msg_build(N, D)
  s1 = msg(g1, src3, dst3)

  g2 = pl.pallas_call(
      _tc2,
      out_shape=jax.ShapeDtypeStruct((N + GPAD, D), jnp.float32),
  )(s1, g1, dinv, b1.reshape(1, D), W2)

  s2 = msg(g2, src3, dst3)

  out = pl.pallas_call(
      _tc3,
      out_shape=jax.ShapeDtypeStruct((N, 1), jnp.float32),
  )(s2, g2, dinv, b2.reshape(1, D), Wf1, bf1.reshape(1, F), Wf2,
    bf2.reshape(1, 1))
  return out


# speedup vs baseline: 1.2383x; 1.1522x over previous
import functools

import jax
import jax.numpy as jnp
from jax import lax
from jax.experimental import pallas as pl
from jax.experimental.pallas import tpu as pltpu
from jax.experimental.pallas import tpu_sc as plsc

NC = 2
NS = 16
NW = NC * NS
L = 16
EB = 40
EPAD = 327680
RW = EPAD // (NW * EB)
CH = 32
RW0 = RW
RW1 = 2 * RW - RW0
GPAD = 16
JPAD = 112


def _mesh():
  return plsc.VectorSubcoreMesh(core_axis_name="c", subcore_axis_name="s")


def _deg_build(N):
  NV = EPAD // NW // L
  DCH = 2048
  NCHUNK = EPAD // NW // DCH
  ND = N + JPAD

  @functools.partial(
      pl.kernel,
      out_type=jax.ShapeDtypeStruct((NW, ND), jnp.float32),
      mesh=_mesh(),
      compiler_params=pltpu.CompilerParams(needs_layout_passes=False),
      scratch_types=[
          pltpu.VMEM((DCH,), jnp.int32),
          pltpu.VMEM((ND,), jnp.float32),
      ],
  )
  def deg_kernel(dst_hbm, out_hbm, idx_v, deg_v):
    cid = lax.axis_index("c")
    sid = lax.axis_index("s")
    wid = sid * NC + cid

    zv = jnp.zeros((L,), jnp.float32)

    def zbody(i, carry):
      deg_v[pl.ds(i * L, L)] = zv
      return carry

    lax.fori_loop(0, ND // L, zbody, 0)

    ones = jnp.ones((L,), jnp.float32)

    def cbody(c, carry):
      pltpu.sync_copy(dst_hbm.at[wid, pl.ds(c * DCH, DCH)], idx_v)

      def ebody(j, carry2):
        idx = idx_v[pl.ds(j * L, L)]
        plsc.addupdate_scatter(deg_v, [idx], ones)
        return carry2

      lax.fori_loop(0, DCH // L, ebody, 0)
      return carry

    lax.fori_loop(0, NCHUNK, cbody, 0)
    pltpu.sync_copy(deg_v, out_hbm.at[wid])

  return deg_kernel


def _msg_build(N, D):
  RS = 8 * ((N + JPAD + 8 * NS - 1) // (8 * NS))
  NP = RS * NS
  ZR = 8

  @functools.partial(
      pl.kernel,
      out_type=jax.ShapeDtypeStruct((NC, NP, D), jnp.float32),
      mesh=_mesh(),
      compiler_params=pltpu.CompilerParams(needs_layout_passes=False),
      scratch_types=[
          pltpu.VMEM((2, CH, EB), jnp.int32),
          pltpu.VMEM((2, CH, EB), jnp.int32),
          pltpu.VMEM((4, EB, D), jnp.float32),
          pltpu.VMEM((ZR, D), jnp.float32),
          pltpu.VMEM_SHARED((NP, D), jnp.float32),
          [pltpu.SemaphoreType.DMA] * 4,
          [pltpu.SemaphoreType.DMA] * 4,
          [pltpu.SemaphoreType.DMA] * 2,
      ],
  )
  def msg_kernel(g_hbm, src_hbm, dst_hbm, out_hbm,
                 src_v, dst_v, gbuf, zbuf, acc, gsem, ssem, isem):
    cid = lax.axis_index("c")
    sid = lax.axis_index("s")
    rw = RW
    base = (cid * NS + sid) * RW

    zv = jnp.zeros((L,), jnp.float32)

    def zbody(i, carry):
      for kk in range(D // L):
        zbuf[i, pl.ds(kk * L, L)] = zv
      return carry

    lax.fori_loop(0, ZR, zbody, 0)

    def zcopy(t, carry):
      pltpu.sync_copy(zbuf, acc.at[pl.ds(sid * RS + t * ZR, ZR)])
      return carry

    lax.fori_loop(0, RS // ZR, zcopy, 0)
    plsc.subcore_barrier()

    NB = 4
    QPC = CH // NB
    NCHK = rw // CH

    def idx_refs(j):
      c = j // CH
      p = c % 2
      l = j - c * CH
      return src_v.at[p, l], dst_v.at[p, l]

    pltpu.sync_copy(src_hbm.at[pl.ds(base, CH)], src_v.at[0])
    pltpu.sync_copy(dst_hbm.at[pl.ds(base, CH)], dst_v.at[0])
    for b in range(NB):
      sref, _ = idx_refs(b)
      pltpu.async_copy(g_hbm.at[sref], gbuf.at[b], gsem[b])

    def qbody(q, carry):
      j0 = NB * q
      c = q // QPC
      qr = q - c * QPC

      @pl.when(jnp.logical_and(qr == 0, c + 1 < NCHK))
      def _():
        cp = (c + 1) % 2
        pltpu.async_copy(src_hbm.at[pl.ds(base + (c + 1) * CH, CH)],
                         src_v.at[cp], isem[0])
        pltpu.async_copy(dst_hbm.at[pl.ds(base + (c + 1) * CH, CH)],
                         dst_v.at[cp], isem[1])

      for b in range(NB):
        sref, dref = idx_refs(j0 + b)
        pltpu.make_async_copy(g_hbm.at[sref], gbuf.at[b], gsem[b]).wait()
        pltpu.async_copy(gbuf.at[b], acc.at[dref], ssem[b], add=True)

      @pl.when(jnp.logical_and(qr == QPC - 1, c + 1 < NCHK))
      def _():
        cp = (c + 1) % 2
        pltpu.make_async_copy(src_hbm.at[pl.ds(base + (c + 1) * CH, CH)],
                              src_v.at[cp], isem[0]).wait()
        pltpu.make_async_copy(dst_hbm.at[pl.ds(base + (c + 1) * CH, CH)],
                              dst_v.at[cp], isem[1]).wait()

      for b in range(NB):
        jn = j0 + NB + b

        @pl.when(jn < rw)
        def _():
          _, dref = idx_refs(j0 + b)
          pltpu.make_async_copy(gbuf.at[b], acc.at[dref], ssem[b]).wait()
          snref, _ = idx_refs(jn)
          pltpu.async_copy(g_hbm.at[snref], gbuf.at[b], gsem[b])

      return carry

    lax.fori_loop(0, rw // NB, qbody, 0)
    for b in range(NB):
      _, dref = idx_refs(rw - NB + b)
      pltpu.make_async_copy(gbuf.at[b], acc.at[dref], ssem[b]).wait()

    plsc.subcore_barrier()
    pltpu.sync_copy(acc.at[pl.ds(sid * RS, RS)],
                    out_hbm.at[cid, pl.ds(sid * RS, RS)])

  return msg_kernel


def _tc1(degp_ref, x_ref, w1_ref, dinv_ref, g1_ref):
  n = x_ref.shape[0]
  deg = 1.0 + jnp.sum(degp_ref[...], axis=0)[:n]
  dinv = lax.rsqrt(deg)[:, None]
  dinv_ref[...] = dinv
  g = jnp.dot(x_ref[...], w1_ref[...],
              preferred_element_type=jnp.float32) * dinv
  g1_ref[...] = jnp.concatenate(
      [g, jnp.zeros((GPAD, g.shape[1]), jnp.float32)], axis=0)


def _tc2(s_ref, g_ref, dinv_ref, b_ref, w_ref, gn_ref):
  n = dinv_ref.shape[0]
  dinv = dinv_ref[...]
  h = jnp.maximum(
      dinv * (s_ref[0, :n] + s_ref[1, :n] + g_ref[:n]) + b_ref[...], 0.0)
  g = jnp.dot(h, w_ref[...], preferred_element_type=jnp.float32) * dinv
  gn_ref[...] = jnp.concatenate(
      [g, jnp.zeros((GPAD, g.shape[1]), jnp.float32)], axis=0)


def _tc3(s_ref, g_ref, dinv_ref, b_ref, wf1_ref, bf1_ref, wf2_ref, bf2_ref,
         out_ref):
  n = dinv_ref.shape[0]
  dinv = dinv_ref[...]
  h2 = jnp.maximum(
      dinv * (s_ref[0, :n] + s_ref[1, :n] + g_ref[:n]) + b_ref[...], 0.0)
  h3 = jnp.maximum(
      jnp.dot(h2, wf1_ref[...], preferred_element_type=jnp.float32)
      + bf1_ref[...], 0.0)
  o = jnp.dot(h3, wf2_ref[...], preferred_element_type=jnp.float32) + bf2_ref[...]
  nrm = jnp.sqrt(jnp.sum(o * o))
  out_ref[...] = o / jnp.maximum(nrm, 1e-12)


def kernel(x, edge_index, W1, b1, W2, b2, Wf1, bf1, Wf2, bf2):
  N, D = x.shape
  E = edge_index.shape[1]
  F = Wf1.shape[1]

  ar = jnp.arange(EPAD - E, dtype=jnp.int32)
  pad_src = (ar * 7) % N
  pad_dst = N + (ar % JPAD)
  src2 = jnp.concatenate([edge_index[0], pad_src]).reshape(NW, RW * EB)
  dst2 = jnp.concatenate([edge_index[1], pad_dst]).reshape(NW, RW * EB)
  src3 = src2.reshape(NW * RW, EB)
  dst3 = dst2.reshape(NW * RW, EB)

  degp = _deg_build(N)(dst2)

  dinv, g1 = pl.pallas_call(
      _tc1,
      out_shape=(jax.ShapeDtypeStruct((N, 1), jnp.float32),
                 jax.ShapeDtypeStruct((N + GPAD, D), jnp.float32)),
  )(degp, x, W1)

  msg = _msg_build(N, D)
  s1 = msg(g1, src3, dst3)

  g2 = pl.pallas_call(
      _tc2,
      out_shape=jax.ShapeDtypeStruct((N + GPAD, D), jnp.float32),
  )(s1, g1, dinv, b1.reshape(1, D), W2)

  s2 = msg(g2, src3, dst3)

  out = pl.pallas_call(
      _tc3,
      out_shape=jax.ShapeDtypeStruct((N, 1), jnp.float32),
  )(s2, g2, dinv, b2.reshape(1, D), Wf1, bf1.reshape(1, F), Wf2,
    bf2.reshape(1, 1))
  return out
